# baseline (speedup 1.0000x reference)
import functools

import jax
import jax.numpy as jnp
from jax import lax
from jax.experimental import pallas as pl
from jax.experimental.pallas import tpu as pltpu
from jax.experimental.pallas import tpu_sc as plsc

N = 10000
E = 320000
D = 128

NC = 2
NS = 16
NW = NC * NS
EPW = E // NW
K = 128
NBLK = 80
EPWP = NBLK * K
PHASES = 2
PBLK = NBLK // PHASES
GRP = 4
NGRP = NBLK // GRP
N_PAD = 10008
N_PAD_DEG = 10240
ZCH = 72
NCHK_Z = N_PAD // ZCH
ZCH_DEG = 128
NCHK_ZD = N_PAD_DEG // ZCH_DEG
CH = 80
NCHK_D = N // CH
CPT_Z = (NCHK_Z + NS - 1) // NS
CPT_ZD = (NCHK_ZD + NS - 1) // NS
CPT_D = (NCHK_D + NS - 1) // NS


def _fill(buf, nrows, vec):
    def fill_row(i, _):
        for j in range(8):
            buf[i, pl.ds(j * 16, 16)] = vec
        return 0

    lax.fori_loop(0, nrows, fill_row, 0)


def _zero_acc(s, acc_sh, zbuf, zsem, zch, nchk, cpt):
    def zero_chunk(t, _):
        chunk = jnp.minimum(s + t * NS, nchk - 1)
        pltpu.async_copy(zbuf, acc_sh.at[pl.ds(chunk * zch, zch)], zsem)
        return 0

    lax.fori_loop(0, cpt, zero_chunk, 0)

    def zero_wait(t, _):
        pltpu.make_async_copy(zbuf, acc_sh.at[pl.ds(0, zch)], zsem).wait()
        return 0

    lax.fori_loop(0, cpt, zero_wait, 0)


def _drain_acc(c, s, acc_sh, out_hbm, zsem):
    def drain_chunk(t, _):
        chunk = jnp.minimum(s + t * NS, NCHK_D - 1)
        r0 = chunk * CH
        pltpu.async_copy(acc_sh.at[pl.ds(r0, CH)],
                         out_hbm.at[c, pl.ds(r0, CH)], zsem)
        return 0

    lax.fori_loop(0, CPT_D, drain_chunk, 0)

    def drain_wait(t, _):
        pltpu.make_async_copy(acc_sh.at[pl.ds(0, CH)],
                              out_hbm.at[c, pl.ds(0, CH)], zsem).wait()
        return 0

    lax.fori_loop(0, CPT_D, drain_wait, 0)


def _sc_agg_body(h_hbm, sd_hbm, out_hbm, acc_sh, sd, rows, gsem, ssem, zsem):
    c = lax.axis_index("c")
    s = lax.axis_index("s")
    wid = s * NC + c

    _fill(rows.at[1], ZCH, jnp.zeros((16,), jnp.float32))
    _zero_acc(s, acc_sh, rows.at[1].at[pl.ds(0, ZCH)], zsem, ZCH, NCHK_Z,
              CPT_Z)
    plsc.subcore_barrier()

    def fire_gather(j, i):
        pltpu.async_copy(h_hbm.at[sd.at[j]], rows.at[i], gsem.at[i])

    def wait_gather(i):
        pltpu.make_async_copy(h_hbm.at[sd.at[0]], rows.at[i],
                              gsem.at[i]).wait()

    def fire_scatter(j, i):
        pltpu.async_copy(rows.at[i], acc_sh.at[sd.at[GRP + j]], ssem.at[i],
                         add=True)

    def wait_scatter(i):
        pltpu.make_async_copy(rows.at[i], acc_sh.at[sd.at[GRP]],
                              ssem.at[i]).wait()

    def grp_body(g, _):
        pltpu.async_copy(sd_hbm.at[wid, g], sd, zsem)
        pltpu.make_async_copy(sd_hbm.at[wid, 0], sd, zsem).wait()
        fire_gather(0, 0)
        fire_gather(1, 1)
        wait_gather(0)
        fire_scatter(0, 0)
        fire_gather(2, 2)
        wait_gather(1)
        fire_scatter(1, 1)
        wait_scatter(0)
        fire_gather(3, 0)
        wait_gather(2)
        fire_scatter(2, 2)
        wait_gather(0)
        fire_scatter(3, 0)
        wait_scatter(1)
        wait_scatter(2)
        wait_scatter(0)
        return 0

    lax.fori_loop(0, NGRP, grp_body, 0)

    plsc.subcore_barrier()
    _drain_acc(c, s, acc_sh, out_hbm, zsem)


def _sc_deg_body(dst_hbm, out_hbm, acc_sh, didx, ones, zbuf, ssem, zsem):
    c = lax.axis_index("c")
    s = lax.axis_index("s")
    wid = s * NC + c

    _fill(zbuf, ZCH_DEG, jnp.zeros((16,), jnp.float32))
    _zero_acc(s, acc_sh, zbuf, zsem, ZCH_DEG, NCHK_ZD, CPT_ZD)
    _fill(ones, K, jnp.ones((16,), jnp.float32))
    plsc.subcore_barrier()

    def fire_scatter(b):
        pltpu.async_copy(ones, acc_sh.at[didx.at[b]], ssem, add=True)

    def wait_scatter():
        pltpu.make_async_copy(ones, acc_sh.at[didx.at[0]], ssem).wait()

    for p in range(PHASES):
        pltpu.async_copy(dst_hbm.at[wid, pl.ds(p * PBLK, PBLK)], didx, zsem)
        pltpu.make_async_copy(dst_hbm.at[wid, pl.ds(0, PBLK)], didx,
                              zsem).wait()

        for i in range(4):
            fire_scatter(i)

        def blk_body(q, _):
            fire_scatter(2 * q + 4)
            fire_scatter(2 * q + 5)
            wait_scatter()
            wait_scatter()
            return 0

        lax.fori_loop(0, (PBLK - 4) // 2, blk_body, 0)
        for i in range(4):
            wait_scatter()

    plsc.subcore_barrier()
    _drain_acc(c, s, acc_sh, out_hbm, zsem)


_MESH = plsc.VectorSubcoreMesh(core_axis_name="c", subcore_axis_name="s",
                               num_cores=NC, num_subcores=NS)


def _sc_aggregate(h, sd_hbm):
    kern = pl.kernel(
        _sc_agg_body,
        out_type=jax.ShapeDtypeStruct((NC, N, D), jnp.float32),
        mesh=_MESH,
        scratch_types=[
            pltpu.VMEM_SHARED((N_PAD, D), jnp.float32),
            pltpu.VMEM((2 * GRP, K), jnp.int32),
            pltpu.VMEM((3, K, D), jnp.float32),
            pltpu.SemaphoreType.DMA((3,)),
            pltpu.SemaphoreType.DMA((3,)),
            pltpu.SemaphoreType.DMA,
        ],
        name="sage_sc_agg",
    )
    return kern(h, sd_hbm)


def _sc_degree(dst3):
    kern = pl.kernel(
        _sc_deg_body,
        out_type=jax.ShapeDtypeStruct((NC, N, D), jnp.float32),
        mesh=_MESH,
        scratch_types=[
            pltpu.VMEM_SHARED((N_PAD_DEG, D), jnp.float32),
            pltpu.VMEM((PBLK, K), jnp.int32),
            pltpu.VMEM((K, D), jnp.float32),
            pltpu.VMEM((ZCH_DEG, D), jnp.float32),
            pltpu.SemaphoreType.DMA,
            pltpu.SemaphoreType.DMA,
        ],
        name="sage_sc_deg",
    )
    return kern(dst3)


def _tc_layer_body(relu, h_ref, a0_ref, a1_ref, d0_ref, d1_ref, ws_ref,
                   wn_ref, b_ref, o_ref):
    deg = jnp.maximum(d0_ref[:, 0:1] + d1_ref[:, 0:1], 1.0)
    mean = (a0_ref[...] + a1_ref[...]) / deg
    out = (jnp.dot(h_ref[...], ws_ref[...], preferred_element_type=jnp.float32)
           + jnp.dot(mean, wn_ref[...], preferred_element_type=jnp.float32)
           + b_ref[...])
    if relu:
        out = jnp.maximum(out, 0.0)
    o_ref[...] = out


def _tc_layer(h, A, degp, Ws, Wn, b, relu):
    F = Ws.shape[1]
    BN = 1000
    grid = (N // BN,)
    out = pl.pallas_call(
        functools.partial(_tc_layer_body, relu),
        grid=grid,
        in_specs=[
            pl.BlockSpec((BN, D), lambda i: (i, 0)),
            pl.BlockSpec((BN, D), lambda i: (i, 0)),
            pl.BlockSpec((BN, D), lambda i: (i, 0)),
            pl.BlockSpec((BN, D), lambda i: (i, 0)),
            pl.BlockSpec((BN, D), lambda i: (i, 0)),
            pl.BlockSpec((D, F), lambda i: (0, 0)),
            pl.BlockSpec((D, F), lambda i: (0, 0)),
            pl.BlockSpec((1, F), lambda i: (0, 0)),
        ],
        out_specs=pl.BlockSpec((BN, F), lambda i: (i, 0)),
        out_shape=jax.ShapeDtypeStruct((N, F), jnp.float32),
        name="sage_tc_layer",
    )(h, A[0], A[1], degp[0], degp[1], Ws, Wn, b.reshape(1, F))
    return out


def kernel(x, edge_index, Wn1, Ws1, b1, Wn2, Ws2, b2, Wn3, Ws3, b3):
    srcw = edge_index[0].reshape(NW, EPW)
    dstw = edge_index[1].reshape(NW, EPW)
    pad_n = EPWP - EPW
    junk = N + (jnp.arange(pad_n, dtype=jnp.int32) % (N_PAD - N))
    srcp = jnp.concatenate(
        [srcw, jnp.zeros((NW, pad_n), jnp.int32)], axis=1)
    dstp = jnp.concatenate(
        [dstw, jnp.broadcast_to(junk, (NW, pad_n))], axis=1)
    sd_hbm = jnp.concatenate(
        [srcp.reshape(NW, NGRP, GRP, K), dstp.reshape(NW, NGRP, GRP, K)],
        axis=2)
    dst3 = dstp.reshape(NW, NBLK, K)
    degp = _sc_degree(dst3)
    A1 = _sc_aggregate(x, sd_hbm)
    h1 = _tc_layer(x, A1, degp, Ws1, Wn1, b1, relu=True)
    A2 = _sc_aggregate(h1, sd_hbm)
    h2 = _tc_layer(h1, A2, degp, Ws2, Wn2, b2, relu=True)
    A3 = _sc_aggregate(h2, sd_hbm)
    out = _tc_layer(h2, A3, degp, Ws3, Wn3, b3, relu=False)
    return out


# speedup vs baseline: 1.3143x; 1.3143x over previous
import functools

import jax
import jax.numpy as jnp
from jax import lax
from jax.experimental import pallas as pl
from jax.experimental.pallas import tpu as pltpu
from jax.experimental.pallas import tpu_sc as plsc

N = 10000
E = 320000
D = 128

NC = 2
NS = 16
NW = NC * NS
EPW = E // NW
K = 128
NBLK = 80
EPWP = NBLK * K
PHASES = 2
PBLK = NBLK // PHASES
KA = 80
NBLKA = EPW // KA
N_PAD_DEG = 10240
ZCH = 8
NCHK_Z = N // ZCH
ZCH_DEG = 128
NCHK_ZD = N_PAD_DEG // ZCH_DEG
CH = 80
NCHK_D = N // CH
CPT_Z = (NCHK_Z + NS - 1) // NS
CPT_ZD = (NCHK_ZD + NS - 1) // NS
CPT_D = (NCHK_D + NS - 1) // NS


def _fill(buf, nrows, vec):
    def fill_row(i, _):
        for j in range(8):
            buf[i, pl.ds(j * 16, 16)] = vec
        return 0

    lax.fori_loop(0, nrows, fill_row, 0)


def _zero_acc(s, acc_sh, zbuf, zsem, zch, nchk, cpt):
    def zero_chunk(t, _):
        chunk = jnp.minimum(s + t * NS, nchk - 1)
        pltpu.async_copy(zbuf, acc_sh.at[pl.ds(chunk * zch, zch)], zsem)
        return 0

    lax.fori_loop(0, cpt, zero_chunk, 0)

    def zero_wait(t, _):
        pltpu.make_async_copy(zbuf, acc_sh.at[pl.ds(0, zch)], zsem).wait()
        return 0

    lax.fori_loop(0, cpt, zero_wait, 0)


def _drain_acc(c, s, acc_sh, out_hbm, zsem):
    def drain_chunk(t, _):
        chunk = jnp.minimum(s + t * NS, NCHK_D - 1)
        r0 = chunk * CH
        pltpu.async_copy(acc_sh.at[pl.ds(r0, CH)],
                         out_hbm.at[c, pl.ds(r0, CH)], zsem)
        return 0

    lax.fori_loop(0, CPT_D, drain_chunk, 0)

    def drain_wait(t, _):
        pltpu.make_async_copy(acc_sh.at[pl.ds(0, CH)],
                              out_hbm.at[c, pl.ds(0, CH)], zsem).wait()
        return 0

    lax.fori_loop(0, CPT_D, drain_wait, 0)


def _sc_agg_body(h_hbm, src_hbm, dst_hbm, out_hbm, acc_sh, sidx, didx, rows,
                 zbuf, sem):
    c = lax.axis_index("c")
    s = lax.axis_index("s")
    wid = s * NC + c

    _fill(zbuf, ZCH, jnp.zeros((16,), jnp.float32))
    _zero_acc(s, acc_sh, zbuf, sem, ZCH, NCHK_Z, CPT_Z)
    plsc.subcore_barrier()

    ebase = wid * EPW

    def edge_blk(b, _):
        off = ebase + b * KA
        pltpu.sync_copy(src_hbm.at[pl.ds(off, KA)], sidx)
        pltpu.sync_copy(dst_hbm.at[pl.ds(off, KA)], didx)
        pltpu.async_copy(h_hbm.at[sidx], rows, sem).wait()
        pltpu.sync_copy(rows, acc_sh.at[didx], add=True)
        return 0

    lax.fori_loop(0, NBLKA, edge_blk, 0)
    plsc.subcore_barrier()

    _drain_acc(c, s, acc_sh, out_hbm, sem)


def _sc_deg_body(dst_hbm, out_hbm, acc_sh, didx, ones, zbuf, ssem, zsem):
    c = lax.axis_index("c")
    s = lax.axis_index("s")
    wid = s * NC + c

    _fill(zbuf, ZCH_DEG, jnp.zeros((16,), jnp.float32))
    _zero_acc(s, acc_sh, zbuf, zsem, ZCH_DEG, NCHK_ZD, CPT_ZD)
    _fill(ones, K, jnp.ones((16,), jnp.float32))
    plsc.subcore_barrier()

    def fire_scatter(b):
        pltpu.async_copy(ones, acc_sh.at[didx.at[b]], ssem, add=True)

    def wait_scatter():
        pltpu.make_async_copy(ones, acc_sh.at[didx.at[0]], ssem).wait()

    for p in range(PHASES):
        pltpu.async_copy(dst_hbm.at[wid, pl.ds(p * PBLK, PBLK)], didx, zsem)
        pltpu.make_async_copy(dst_hbm.at[wid, pl.ds(0, PBLK)], didx,
                              zsem).wait()

        for i in range(4):
            fire_scatter(i)

        def blk_body(q, _):
            fire_scatter(2 * q + 4)
            fire_scatter(2 * q + 5)
            wait_scatter()
            wait_scatter()
            return 0

        lax.fori_loop(0, (PBLK - 4) // 2, blk_body, 0)
        for i in range(4):
            wait_scatter()

    plsc.subcore_barrier()
    _drain_acc(c, s, acc_sh, out_hbm, zsem)


_MESH = plsc.VectorSubcoreMesh(core_axis_name="c", subcore_axis_name="s",
                               num_cores=NC, num_subcores=NS)


def _sc_aggregate(h, src, dst):
    kern = pl.kernel(
        _sc_agg_body,
        out_type=jax.ShapeDtypeStruct((NC, N, D), jnp.float32),
        mesh=_MESH,
        scratch_types=[
            pltpu.VMEM_SHARED((N, D), jnp.float32),
            pltpu.VMEM((KA,), jnp.int32),
            pltpu.VMEM((KA,), jnp.int32),
            pltpu.VMEM((KA, D), jnp.float32),
            pltpu.VMEM((ZCH, D), jnp.float32),
            pltpu.SemaphoreType.DMA,
        ],
        name="sage_sc_agg",
    )
    return kern(h, src, dst)


def _sc_degree(dst3):
    kern = pl.kernel(
        _sc_deg_body,
        out_type=jax.ShapeDtypeStruct((NC, N, D), jnp.float32),
        mesh=_MESH,
        scratch_types=[
            pltpu.VMEM_SHARED((N_PAD_DEG, D), jnp.float32),
            pltpu.VMEM((PBLK, K), jnp.int32),
            pltpu.VMEM((K, D), jnp.float32),
            pltpu.VMEM((ZCH_DEG, D), jnp.float32),
            pltpu.SemaphoreType.DMA,
            pltpu.SemaphoreType.DMA,
        ],
        name="sage_sc_deg",
    )
    return kern(dst3)


def _tc_layer_body(relu, h_ref, a0_ref, a1_ref, d0_ref, d1_ref, ws_ref,
                   wn_ref, b_ref, o_ref):
    deg = jnp.maximum(d0_ref[:, 0:1] + d1_ref[:, 0:1], 1.0)
    mean = (a0_ref[...] + a1_ref[...]) / deg
    out = (jnp.dot(h_ref[...], ws_ref[...], preferred_element_type=jnp.float32)
           + jnp.dot(mean, wn_ref[...], preferred_element_type=jnp.float32)
           + b_ref[...])
    if relu:
        out = jnp.maximum(out, 0.0)
    o_ref[...] = out


def _tc_layer(h, A, degp, Ws, Wn, b, relu):
    F = Ws.shape[1]
    BN = 1000
    grid = (N // BN,)
    out = pl.pallas_call(
        functools.partial(_tc_layer_body, relu),
        grid=grid,
        in_specs=[
            pl.BlockSpec((BN, D), lambda i: (i, 0)),
            pl.BlockSpec((BN, D), lambda i: (i, 0)),
            pl.BlockSpec((BN, D), lambda i: (i, 0)),
            pl.BlockSpec((BN, D), lambda i: (i, 0)),
            pl.BlockSpec((BN, D), lambda i: (i, 0)),
            pl.BlockSpec((D, F), lambda i: (0, 0)),
            pl.BlockSpec((D, F), lambda i: (0, 0)),
            pl.BlockSpec((1, F), lambda i: (0, 0)),
        ],
        out_specs=pl.BlockSpec((BN, F), lambda i: (i, 0)),
        out_shape=jax.ShapeDtypeStruct((N, F), jnp.float32),
        name="sage_tc_layer",
    )(h, A[0], A[1], degp[0], degp[1], Ws, Wn, b.reshape(1, F))
    return out


def kernel(x, edge_index, Wn1, Ws1, b1, Wn2, Ws2, b2, Wn3, Ws3, b3):
    src = edge_index[0]
    dst = edge_index[1]
    dstw = dst.reshape(NW, EPW)
    pad_n = EPWP - EPW
    junk = N + (jnp.arange(pad_n, dtype=jnp.int32) % (N_PAD_DEG - N))
    dst3 = jnp.concatenate(
        [dstw, jnp.broadcast_to(junk, (NW, pad_n))], axis=1).reshape(
            NW, NBLK, K)
    degp = _sc_degree(dst3)
    A1 = _sc_aggregate(x, src, dst)
    h1 = _tc_layer(x, A1, degp, Ws1, Wn1, b1, relu=True)
    A2 = _sc_aggregate(h1, src, dst)
    h2 = _tc_layer(h1, A2, degp, Ws2, Wn2, b2, relu=True)
    A3 = _sc_aggregate(h2, src, dst)
    out = _tc_layer(h2, A3, degp, Ws3, Wn3, b3, relu=False)
    return out


# speedup vs baseline: 2.7541x; 2.0954x over previous
import functools

import jax
import jax.numpy as jnp
from jax import lax
from jax.experimental import pallas as pl
from jax.experimental.pallas import tpu as pltpu
from jax.experimental.pallas import tpu_sc as plsc

N = 10000
E = 320000
D = 128

NC = 2
NS = 16
NW = NC * NS
EPW = E // NW
K = 128
NBLK = 80
EPWP = NBLK * K
PHASES = 2
PBLK = NBLK // PHASES
KA = 80
NBLKA = EPW // KA
N_PAD_DEG = 10240
ZCH = 8
NCHK_Z = N // ZCH
ZCH_DEG = 128
NCHK_ZD = N_PAD_DEG // ZCH_DEG
CH = 80
NCHK_D = N // CH
CPT_Z = (NCHK_Z + NS - 1) // NS
CPT_ZD = (NCHK_ZD + NS - 1) // NS
CPT_D = (NCHK_D + NS - 1) // NS


def _fill(buf, nrows, vec):
    def fill_row(i, _):
        for j in range(8):
            buf[i, pl.ds(j * 16, 16)] = vec
        return 0

    lax.fori_loop(0, nrows, fill_row, 0)


def _zero_acc(s, acc_sh, zbuf, zsem, zch, nchk, cpt):
    def zero_chunk(t, _):
        chunk = jnp.minimum(s + t * NS, nchk - 1)
        pltpu.async_copy(zbuf, acc_sh.at[pl.ds(chunk * zch, zch)], zsem)
        return 0

    lax.fori_loop(0, cpt, zero_chunk, 0)

    def zero_wait(t, _):
        pltpu.make_async_copy(zbuf, acc_sh.at[pl.ds(0, zch)], zsem).wait()
        return 0

    lax.fori_loop(0, cpt, zero_wait, 0)


def _drain_acc(c, s, acc_sh, out_hbm, zsem):
    def drain_chunk(t, _):
        chunk = jnp.minimum(s + t * NS, NCHK_D - 1)
        r0 = chunk * CH
        pltpu.async_copy(acc_sh.at[pl.ds(r0, CH)],
                         out_hbm.at[c, pl.ds(r0, CH)], zsem)
        return 0

    lax.fori_loop(0, CPT_D, drain_chunk, 0)

    def drain_wait(t, _):
        pltpu.make_async_copy(acc_sh.at[pl.ds(0, CH)],
                              out_hbm.at[c, pl.ds(0, CH)], zsem).wait()
        return 0

    lax.fori_loop(0, CPT_D, drain_wait, 0)


def _sc_agg_body(h_hbm, src_hbm, dst_hbm, out_hbm, acc_sh, sidx, didx, rows,
                 zbuf, gsem, isem, sem):
    c = lax.axis_index("c")
    s = lax.axis_index("s")
    wid = s * NC + c

    _fill(zbuf, ZCH, jnp.zeros((16,), jnp.float32))
    _zero_acc(s, acc_sh, zbuf, sem, ZCH, NCHK_Z, CPT_Z)
    plsc.subcore_barrier()

    ebase = wid * EPW

    def fire_idx(b, sl):
        off = ebase + jnp.minimum(b, NBLKA - 1) * KA
        pltpu.async_copy(src_hbm.at[pl.ds(off, KA)], sidx.at[sl], isem.at[sl])
        pltpu.async_copy(dst_hbm.at[pl.ds(off, KA)], didx.at[sl], isem.at[sl])

    def wait_idx(sl):
        pltpu.make_async_copy(src_hbm.at[pl.ds(0, KA)], sidx.at[sl],
                              isem.at[sl]).wait()
        pltpu.make_async_copy(dst_hbm.at[pl.ds(0, KA)], didx.at[sl],
                              isem.at[sl]).wait()

    def fire_gather(sl, r):
        pltpu.async_copy(h_hbm.at[sidx.at[sl]], rows.at[r], gsem.at[r])

    def wait_gather(r):
        pltpu.make_async_copy(h_hbm.at[sidx.at[0]], rows.at[r],
                              gsem.at[r]).wait()

    def scatter(sl, r):
        pltpu.sync_copy(rows.at[r], acc_sh.at[didx.at[sl]], add=True)

    for sl in range(4):
        fire_idx(sl, sl)
    wait_idx(0)
    fire_gather(0, 0)
    wait_idx(1)
    fire_gather(1, 1)

    def iter_body(q, _):
        b0 = q * 4
        for j in range(4):
            r = j % 2
            wait_gather(r)
            scatter(j, r)
            fire_idx(b0 + j + 4, j)
            nxt = (j + 2) % 4
            wait_idx(nxt)
            fire_gather(nxt, r)
        return 0

    lax.fori_loop(0, (NBLKA - 1) // 4, iter_body, 0)

    wait_gather(0)
    scatter(0, 0)
    wait_gather(1)
    wait_idx(2)
    wait_idx(3)

    plsc.subcore_barrier()
    _drain_acc(c, s, acc_sh, out_hbm, sem)


def _sc_deg_body(dst_hbm, out_hbm, acc_sh, didx, ones, zbuf, ssem, zsem):
    c = lax.axis_index("c")
    s = lax.axis_index("s")
    wid = s * NC + c

    _fill(zbuf, ZCH_DEG, jnp.zeros((16,), jnp.float32))
    _zero_acc(s, acc_sh, zbuf, zsem, ZCH_DEG, NCHK_ZD, CPT_ZD)
    _fill(ones, K, jnp.ones((16,), jnp.float32))
    plsc.subcore_barrier()

    def fire_scatter(b):
        pltpu.async_copy(ones, acc_sh.at[didx.at[b]], ssem, add=True)

    def wait_scatter():
        pltpu.make_async_copy(ones, acc_sh.at[didx.at[0]], ssem).wait()

    for p in range(PHASES):
        pltpu.async_copy(dst_hbm.at[wid, pl.ds(p * PBLK, PBLK)], didx, zsem)
        pltpu.make_async_copy(dst_hbm.at[wid, pl.ds(0, PBLK)], didx,
                              zsem).wait()

        for i in range(4):
            fire_scatter(i)

        def blk_body(q, _):
            fire_scatter(2 * q + 4)
            fire_scatter(2 * q + 5)
            wait_scatter()
            wait_scatter()
            return 0

        lax.fori_loop(0, (PBLK - 4) // 2, blk_body, 0)
        for i in range(4):
            wait_scatter()

    plsc.subcore_barrier()
    _drain_acc(c, s, acc_sh, out_hbm, zsem)


_MESH = plsc.VectorSubcoreMesh(core_axis_name="c", subcore_axis_name="s",
                               num_cores=NC, num_subcores=NS)


def _sc_aggregate(h, src, dst):
    kern = pl.kernel(
        _sc_agg_body,
        out_type=jax.ShapeDtypeStruct((NC, N, D), jnp.float32),
        mesh=_MESH,
        scratch_types=[
            pltpu.VMEM_SHARED((N, D), jnp.float32),
            pltpu.VMEM((4, KA), jnp.int32),
            pltpu.VMEM((4, KA), jnp.int32),
            pltpu.VMEM((2, KA, D), jnp.float32),
            pltpu.VMEM((ZCH, D), jnp.float32),
            pltpu.SemaphoreType.DMA((2,)),
            pltpu.SemaphoreType.DMA((4,)),
            pltpu.SemaphoreType.DMA,
        ],
        name="sage_sc_agg",
    )
    return kern(h, src, dst)


def _sc_degree(dst3):
    kern = pl.kernel(
        _sc_deg_body,
        out_type=jax.ShapeDtypeStruct((NC, N, D), jnp.float32),
        mesh=_MESH,
        scratch_types=[
            pltpu.VMEM_SHARED((N_PAD_DEG, D), jnp.float32),
            pltpu.VMEM((PBLK, K), jnp.int32),
            pltpu.VMEM((K, D), jnp.float32),
            pltpu.VMEM((ZCH_DEG, D), jnp.float32),
            pltpu.SemaphoreType.DMA,
            pltpu.SemaphoreType.DMA,
        ],
        name="sage_sc_deg",
    )
    return kern(dst3)


def _tc_layer_body(relu, h_ref, a0_ref, a1_ref, d0_ref, d1_ref, ws_ref,
                   wn_ref, b_ref, o_ref):
    deg = jnp.maximum(d0_ref[:, 0:1] + d1_ref[:, 0:1], 1.0)
    mean = (a0_ref[...] + a1_ref[...]) / deg
    out = (jnp.dot(h_ref[...], ws_ref[...], preferred_element_type=jnp.float32)
           + jnp.dot(mean, wn_ref[...], preferred_element_type=jnp.float32)
           + b_ref[...])
    if relu:
        out = jnp.maximum(out, 0.0)
    o_ref[...] = out


def _tc_layer(h, A, degp, Ws, Wn, b, relu):
    F = Ws.shape[1]
    BN = 1000
    grid = (N // BN,)
    out = pl.pallas_call(
        functools.partial(_tc_layer_body, relu),
        grid=grid,
        in_specs=[
            pl.BlockSpec((BN, D), lambda i: (i, 0)),
            pl.BlockSpec((BN, D), lambda i: (i, 0)),
            pl.BlockSpec((BN, D), lambda i: (i, 0)),
            pl.BlockSpec((BN, D), lambda i: (i, 0)),
            pl.BlockSpec((BN, D), lambda i: (i, 0)),
            pl.BlockSpec((D, F), lambda i: (0, 0)),
            pl.BlockSpec((D, F), lambda i: (0, 0)),
            pl.BlockSpec((1, F), lambda i: (0, 0)),
        ],
        out_specs=pl.BlockSpec((BN, F), lambda i: (i, 0)),
        out_shape=jax.ShapeDtypeStruct((N, F), jnp.float32),
        name="sage_tc_layer",
    )(h, A[0], A[1], degp[0], degp[1], Ws, Wn, b.reshape(1, F))
    return out


def kernel(x, edge_index, Wn1, Ws1, b1, Wn2, Ws2, b2, Wn3, Ws3, b3):
    src = edge_index[0]
    dst = edge_index[1]
    dstw = dst.reshape(NW, EPW)
    pad_n = EPWP - EPW
    junk = N + (jnp.arange(pad_n, dtype=jnp.int32) % (N_PAD_DEG - N))
    dst3 = jnp.concatenate(
        [dstw, jnp.broadcast_to(junk, (NW, pad_n))], axis=1).reshape(
            NW, NBLK, K)
    degp = _sc_degree(dst3)
    A1 = _sc_aggregate(x, src, dst)
    h1 = _tc_layer(x, A1, degp, Ws1, Wn1, b1, relu=True)
    A2 = _sc_aggregate(h1, src, dst)
    h2 = _tc_layer(h1, A2, degp, Ws2, Wn2, b2, relu=True)
    A3 = _sc_aggregate(h2, src, dst)
    out = _tc_layer(h2, A3, degp, Ws3, Wn3, b3, relu=False)
    return out
